# baseline (speedup 1.0000x reference)
import functools

import jax
import jax.numpy as jnp
from jax import lax
from jax.experimental import pallas as pl
from jax.experimental.pallas import tpu as pltpu
from jax.experimental.pallas import tpu_sc as plsc

_W_ADJ = 0.4
_INV_R_ADJ = 0.1
_BV = 8192
_VPAD = 106496
_B = 16384
_L = 200
_NTILES = 32
_ROWS_PER_W = _B // _NTILES
_CHUNK = 32
_NCHUNK = _ROWS_PER_W // _CHUNK
_UNROLL = 8


def _fuse_body(w_ref, r0_ref, r1_ref, t_ref):
    w = (w_ref[...] + _W_ADJ) * _INV_R_ADJ
    u0 = lax.bitcast_convert_type(w * r0_ref[...], jnp.uint32) + 0x8000
    u1 = lax.bitcast_convert_type(w * r1_ref[...], jnp.uint32) + 0x8000
    word = (u1 & jnp.uint32(0xFFFF0000)) | (u0 >> 16)
    t_ref[...] = lax.bitcast_convert_type(word, jnp.int32)


def _fuse_tables(w1, r0, r1):
    return pl.pallas_call(
        _fuse_body,
        grid=(_VPAD // _BV,),
        in_specs=[
            pl.BlockSpec((_BV,), lambda j: (j,)),
            pl.BlockSpec((_BV,), lambda j: (j,)),
            pl.BlockSpec((_BV,), lambda j: (j,)),
        ],
        out_specs=pl.BlockSpec((_BV,), lambda j: (j,)),
        out_shape=jax.ShapeDtypeStruct((_VPAD,), jnp.int32),
    )(w1, r0, r1)


def _sc_body(t_hbm, idx_hbm, out_hbm, table_v, idx_a, idx_b, out_v, sem_a,
             sem_b):
    w = lax.axis_index("s") * 2 + lax.axis_index("c")
    row0 = w * _ROWS_PER_W

    pltpu.sync_copy(t_hbm, table_v)

    lane = lax.iota(jnp.int32, 16)
    hi_mask = jnp.full((16,), -65536, dtype=jnp.int32)

    def chunk_src(chunk):
        return idx_hbm.at[pl.ds((row0 + chunk * _CHUNK) * _L, _CHUNK * _L)]

    def consume(chunk, buf):
        for sub in range(_CHUNK // 16):
            base = (lane + (sub * 16)) * _L

            def tok_body(i, accs):
                a0, a1 = accs
                for k in range(_UNROLL):
                    tok = plsc.load_gather(buf, [base + (i * _UNROLL + k)])
                    word = plsc.load_gather(table_v, [tok])
                    a0 = a0 + plsc.bitcast(word << 16, jnp.float32)
                    a1 = a1 + plsc.bitcast(word & hi_mask, jnp.float32)
                return a0, a1

            zero = jnp.zeros((16,), jnp.float32)
            a0, a1 = lax.fori_loop(0, _L // _UNROLL, tok_body, (zero, zero))
            pos = (lane + (chunk * _CHUNK + sub * 16)) * 2
            plsc.store_scatter(out_v, [pos], a0)
            plsc.store_scatter(out_v, [pos + 1], a1)

    pltpu.async_copy(chunk_src(0), idx_a, sem_a)

    def pair_body(g, _):
        c0 = g * 2
        pltpu.async_copy(chunk_src(c0 + 1), idx_b, sem_b)
        pltpu.make_async_copy(chunk_src(c0), idx_a, sem_a).wait()
        consume(c0, idx_a)

        @pl.when(g < _NCHUNK // 2 - 1)
        def _():
            pltpu.async_copy(chunk_src(c0 + 2), idx_a, sem_a)

        pltpu.make_async_copy(chunk_src(c0 + 1), idx_b, sem_b).wait()
        consume(c0 + 1, idx_b)
        return ()

    lax.fori_loop(0, _NCHUNK // 2, pair_body, ())

    pltpu.sync_copy(out_v, out_hbm.at[pl.ds(row0 * 2, _ROWS_PER_W * 2)])


@functools.partial(
    pl.kernel,
    mesh=plsc.VectorSubcoreMesh(core_axis_name="c", subcore_axis_name="s"),
    out_type=jax.ShapeDtypeStruct((_B * 2,), jnp.float32),
    compiler_params=pltpu.CompilerParams(
        needs_layout_passes=False, use_tc_tiling_on_sc=False),
    scratch_types=[
        pltpu.VMEM((_VPAD,), jnp.int32),
        pltpu.VMEM((_CHUNK * _L,), jnp.int32),
        pltpu.VMEM((_CHUNK * _L,), jnp.int32),
        pltpu.VMEM((_ROWS_PER_W * 2,), jnp.float32),
        pltpu.SemaphoreType.DMA,
        pltpu.SemaphoreType.DMA,
    ],
)
def _sc_gather_sum(t_hbm, idx_hbm, out_hbm, table_v, idx_a, idx_b, out_v,
                   sem_a, sem_b):
    _sc_body(t_hbm, idx_hbm, out_hbm, table_v, idx_a, idx_b, out_v, sem_a,
             sem_b)


def kernel(W, R, feat_idx):
    fused = _fuse_tables(W[:, 0], R[:, 0], R[:, 1])
    flat = _sc_gather_sum(fused, feat_idx.astype(jnp.int32).reshape(-1))
    return flat.reshape(_B, 2)


# speedup vs baseline: 1.1795x; 1.1795x over previous
import functools

import jax
import jax.numpy as jnp
from jax import lax
from jax.experimental import pallas as pl
from jax.experimental.pallas import tpu as pltpu
from jax.experimental.pallas import tpu_sc as plsc

_W_ADJ = 0.4
_INV_R_ADJ = 0.1
_BV = 8192
_VPAD = 106496
_B = 16384
_L = 200
_NTILES = 32
_ROWS_PER_W = _B // _NTILES
_CHUNK = 32
_NCHUNK = _ROWS_PER_W // _CHUNK
_UNROLL = 8


def _fuse_body(w_ref, r0_ref, r1_ref, t_ref):
    w = (w_ref[...] + _W_ADJ) * _INV_R_ADJ
    u0 = lax.bitcast_convert_type(w * r0_ref[...], jnp.uint32) + 0x8000
    u1 = lax.bitcast_convert_type(w * r1_ref[...], jnp.uint32) + 0x8000
    word = (u1 & jnp.uint32(0xFFFF0000)) | (u0 >> 16)
    t_ref[...] = lax.bitcast_convert_type(word, jnp.int32)


def _fuse_tables(w1, r0, r1):
    return pl.pallas_call(
        _fuse_body,
        grid=(_VPAD // _BV,),
        in_specs=[
            pl.BlockSpec((_BV,), lambda j: (j,)),
            pl.BlockSpec((_BV,), lambda j: (j,)),
            pl.BlockSpec((_BV,), lambda j: (j,)),
        ],
        out_specs=pl.BlockSpec((_BV,), lambda j: (j,)),
        out_shape=jax.ShapeDtypeStruct((_VPAD,), jnp.int32),
    )(w1, r0, r1)


def _sc_body(t_hbm, idx_hbm, out_hbm, table_v, idx_a, idx_b, out0_v, out1_v,
             sem_a, sem_b, sem_t):
    w = lax.axis_index("s") * 2 + lax.axis_index("c")
    row0 = w * _ROWS_PER_W

    lane = lax.iota(jnp.int32, 16)
    hi_mask = jnp.full((16,), -65536, dtype=jnp.int32)

    def chunk_src(chunk):
        return idx_hbm.at[pl.ds((row0 + chunk * _CHUNK) * _L, _CHUNK * _L)]

    def consume(chunk, buf):
        for sub in range(_CHUNK // 16):
            base = (lane + (sub * 16)) * _L

            def tok_body(i, accs):
                a0, a1 = accs
                for k in range(_UNROLL):
                    tok = plsc.load_gather(buf, [base + (i * _UNROLL + k)])
                    word = plsc.load_gather(table_v, [tok])
                    a0 = a0 + plsc.bitcast(word << 16, jnp.float32)
                    a1 = a1 + plsc.bitcast(word & hi_mask, jnp.float32)
                return a0, a1

            zero = jnp.zeros((16,), jnp.float32)
            a0, a1 = lax.fori_loop(0, _L // _UNROLL, tok_body, (zero, zero))
            off = chunk * _CHUNK + sub * 16
            out0_v[pl.ds(off, 16)] = a0
            out1_v[pl.ds(off, 16)] = a1

    pltpu.async_copy(t_hbm, table_v, sem_t)
    pltpu.async_copy(chunk_src(0), idx_a, sem_a)
    pltpu.make_async_copy(t_hbm, table_v, sem_t).wait()

    def pair_body(g, _):
        c0 = g * 2
        pltpu.async_copy(chunk_src(c0 + 1), idx_b, sem_b)
        pltpu.make_async_copy(chunk_src(c0), idx_a, sem_a).wait()
        consume(c0, idx_a)

        @pl.when(g < _NCHUNK // 2 - 1)
        def _():
            pltpu.async_copy(chunk_src(c0 + 2), idx_a, sem_a)

        pltpu.make_async_copy(chunk_src(c0 + 1), idx_b, sem_b).wait()
        consume(c0 + 1, idx_b)
        return ()

    lax.fori_loop(0, _NCHUNK // 2, pair_body, ())

    pltpu.sync_copy(out0_v, out_hbm.at[0, pl.ds(row0, _ROWS_PER_W)])
    pltpu.sync_copy(out1_v, out_hbm.at[1, pl.ds(row0, _ROWS_PER_W)])


@functools.partial(
    pl.kernel,
    mesh=plsc.VectorSubcoreMesh(core_axis_name="c", subcore_axis_name="s"),
    out_type=jax.ShapeDtypeStruct((2, _B), jnp.float32),
    compiler_params=pltpu.CompilerParams(
        needs_layout_passes=False, use_tc_tiling_on_sc=False),
    scratch_types=[
        pltpu.VMEM((_VPAD,), jnp.int32),
        pltpu.VMEM((_CHUNK * _L,), jnp.int32),
        pltpu.VMEM((_CHUNK * _L,), jnp.int32),
        pltpu.VMEM((_ROWS_PER_W,), jnp.float32),
        pltpu.VMEM((_ROWS_PER_W,), jnp.float32),
        pltpu.SemaphoreType.DMA,
        pltpu.SemaphoreType.DMA,
        pltpu.SemaphoreType.DMA,
    ],
)
def _sc_gather_sum(t_hbm, idx_hbm, out_hbm, table_v, idx_a, idx_b, out0_v,
                   out1_v, sem_a, sem_b, sem_t):
    _sc_body(t_hbm, idx_hbm, out_hbm, table_v, idx_a, idx_b, out0_v, out1_v,
             sem_a, sem_b, sem_t)


def kernel(W, R, feat_idx):
    fused = _fuse_tables(W[:, 0], R[:, 0], R[:, 1])
    out2 = _sc_gather_sum(fused, feat_idx.astype(jnp.int32).reshape(-1))
    return out2.T


# speedup vs baseline: 1.1813x; 1.0016x over previous
import functools

import jax
import jax.numpy as jnp
from jax import lax
from jax.experimental import pallas as pl
from jax.experimental.pallas import tpu as pltpu
from jax.experimental.pallas import tpu_sc as plsc

_W_ADJ = 0.4
_INV_R_ADJ = 0.1
_BV = 8192
_VPAD = 106496
_B = 16384
_L = 200
_NTILES = 32
_ROWS_PER_W = _B // _NTILES
_CHUNK = 32
_NCHUNK = _ROWS_PER_W // _CHUNK
_UNROLL = 8


def _fuse_body(w_ref, r0_ref, r1_ref, t_ref):
    w = (w_ref[...] + _W_ADJ) * _INV_R_ADJ
    u0 = lax.bitcast_convert_type(w * r0_ref[...], jnp.uint32) + 0x8000
    u1 = lax.bitcast_convert_type(w * r1_ref[...], jnp.uint32) + 0x8000
    word = (u1 & jnp.uint32(0xFFFF0000)) | (u0 >> 16)
    t_ref[...] = lax.bitcast_convert_type(word, jnp.int32)


def _fuse_tables(w1, r0, r1):
    return pl.pallas_call(
        _fuse_body,
        grid=(_VPAD // _BV,),
        in_specs=[
            pl.BlockSpec((_BV,), lambda j: (j,)),
            pl.BlockSpec((_BV,), lambda j: (j,)),
            pl.BlockSpec((_BV,), lambda j: (j,)),
        ],
        out_specs=pl.BlockSpec((_BV,), lambda j: (j,)),
        out_shape=jax.ShapeDtypeStruct((_VPAD,), jnp.int32),
    )(w1, r0, r1)


def _sc_body(t_hbm, idx_hbm, out_hbm, table_v, idx_a, idx_b, out0_v, out1_v,
             sem_a, sem_b, sem_t):
    w = lax.axis_index("s") * 2 + lax.axis_index("c")
    row0 = w * _ROWS_PER_W

    lane = lax.iota(jnp.int32, 16)
    hi_mask = jnp.full((16,), -65536, dtype=jnp.int32)

    def chunk_src(chunk):
        return idx_hbm.at[pl.ds(row0 + chunk * _CHUNK, _CHUNK)]

    def consume(chunk, buf):
        for sub in range(_CHUNK // 16):
            rows = lane + (sub * 16)

            def tok_body(i, accs):
                a0, a1 = accs
                for k in range(_UNROLL):
                    col = jnp.full((16,), i * _UNROLL + k, dtype=jnp.int32)
                    tok = plsc.load_gather(buf, [rows, col])
                    word = plsc.load_gather(table_v, [tok])
                    a0 = a0 + plsc.bitcast(word << 16, jnp.float32)
                    a1 = a1 + plsc.bitcast(word & hi_mask, jnp.float32)
                return a0, a1

            zero = jnp.zeros((16,), jnp.float32)
            a0, a1 = lax.fori_loop(0, _L // _UNROLL, tok_body, (zero, zero))
            off = chunk * _CHUNK + sub * 16
            out0_v[pl.ds(off, 16)] = a0
            out1_v[pl.ds(off, 16)] = a1

    pltpu.async_copy(t_hbm, table_v, sem_t)
    pltpu.async_copy(chunk_src(0), idx_a, sem_a)
    pltpu.make_async_copy(t_hbm, table_v, sem_t).wait()

    def pair_body(g, _):
        c0 = g * 2
        pltpu.async_copy(chunk_src(c0 + 1), idx_b, sem_b)
        pltpu.make_async_copy(chunk_src(c0), idx_a, sem_a).wait()
        consume(c0, idx_a)

        @pl.when(g < _NCHUNK // 2 - 1)
        def _():
            pltpu.async_copy(chunk_src(c0 + 2), idx_a, sem_a)

        pltpu.make_async_copy(chunk_src(c0 + 1), idx_b, sem_b).wait()
        consume(c0 + 1, idx_b)
        return ()

    lax.fori_loop(0, _NCHUNK // 2, pair_body, ())

    pltpu.sync_copy(out0_v, out_hbm.at[0, pl.ds(row0, _ROWS_PER_W)])
    pltpu.sync_copy(out1_v, out_hbm.at[1, pl.ds(row0, _ROWS_PER_W)])


@functools.partial(
    pl.kernel,
    mesh=plsc.VectorSubcoreMesh(core_axis_name="c", subcore_axis_name="s"),
    out_type=jax.ShapeDtypeStruct((2, _B), jnp.float32),
    compiler_params=pltpu.CompilerParams(
        needs_layout_passes=False, use_tc_tiling_on_sc=False),
    scratch_types=[
        pltpu.VMEM((_VPAD,), jnp.int32),
        pltpu.VMEM((_CHUNK, _L), jnp.int32),
        pltpu.VMEM((_CHUNK, _L), jnp.int32),
        pltpu.VMEM((_ROWS_PER_W,), jnp.float32),
        pltpu.VMEM((_ROWS_PER_W,), jnp.float32),
        pltpu.SemaphoreType.DMA,
        pltpu.SemaphoreType.DMA,
        pltpu.SemaphoreType.DMA,
    ],
)
def _sc_gather_sum(t_hbm, idx_hbm, out_hbm, table_v, idx_a, idx_b, out0_v,
                   out1_v, sem_a, sem_b, sem_t):
    _sc_body(t_hbm, idx_hbm, out_hbm, table_v, idx_a, idx_b, out0_v, out1_v,
             sem_a, sem_b, sem_t)


def kernel(W, R, feat_idx):
    fused = _fuse_tables(W[:, 0], R[:, 0], R[:, 1])
    out2 = _sc_gather_sum(fused, feat_idx.astype(jnp.int32))
    return out2.T


# speedup vs baseline: 1.2910x; 1.0928x over previous
import functools

import jax
import jax.numpy as jnp
from jax import lax
from jax.experimental import pallas as pl
from jax.experimental.pallas import tpu as pltpu
from jax.experimental.pallas import tpu_sc as plsc

_W_ADJ = 0.4
_INV_R_ADJ = 0.1
_BV = 10240
_VPAD = 102400
_B = 16384
_L = 200
_NTILES = 32
_ROWS_PER_W = _B // _NTILES
_LCH = 25
_NCHUNK = _L // _LCH
_UNROLL = 5


def _fuse_body(w_ref, r0_ref, r1_ref, t_ref):
    w = (w_ref[...] + _W_ADJ) * _INV_R_ADJ
    u0 = lax.bitcast_convert_type(w * r0_ref[...], jnp.uint32) + 0x8000
    u1 = lax.bitcast_convert_type(w * r1_ref[...], jnp.uint32) + 0x8000
    word = (u1 & jnp.uint32(0xFFFF0000)) | (u0 >> 16)
    t_ref[...] = lax.bitcast_convert_type(word, jnp.int32)


def _fuse_tables(w1, r0, r1):
    return pl.pallas_call(
        _fuse_body,
        grid=(_VPAD // _BV,),
        in_specs=[
            pl.BlockSpec((_BV,), lambda j: (j,)),
            pl.BlockSpec((_BV,), lambda j: (j,)),
            pl.BlockSpec((_BV,), lambda j: (j,)),
        ],
        out_specs=pl.BlockSpec((_BV,), lambda j: (j,)),
        out_shape=jax.ShapeDtypeStruct((_VPAD,), jnp.int32),
    )(w1, r0, r1)


def _sc_body(t_hbm, idx_hbm, out_hbm, table_v, idx_a, idx_b, out0_v, out1_v,
             sem_a, sem_b, sem_t):
    w = lax.axis_index("s") * 2 + lax.axis_index("c")
    row0 = w * _ROWS_PER_W

    lane = lax.iota(jnp.int32, 16)
    hi_mask = jnp.full((16,), -65536, dtype=jnp.int32)

    def chunk_src(chunk):
        return idx_hbm.at[pl.ds(chunk * _LCH, _LCH), pl.ds(row0, _ROWS_PER_W)]

    zero = jnp.zeros((16,), jnp.float32)
    for g in range(_ROWS_PER_W // 16):
        out0_v[pl.ds(g * 16, 16)] = zero
        out1_v[pl.ds(g * 16, 16)] = zero

    def consume(chunk, buf):
        del chunk
        for g in range(_ROWS_PER_W // 16):
            col0 = g * 16

            def tok_body(i, accs):
                a0, a1 = accs
                for k in range(_UNROLL):
                    tok = buf[i * _UNROLL + k, pl.ds(col0, 16)]
                    word = plsc.load_gather(table_v, [tok])
                    a0 = a0 + plsc.bitcast(word << 16, jnp.float32)
                    a1 = a1 + plsc.bitcast(word & hi_mask, jnp.float32)
                return a0, a1

            a0, a1 = lax.fori_loop(
                0, _LCH // _UNROLL, tok_body,
                (out0_v[pl.ds(col0, 16)], out1_v[pl.ds(col0, 16)]))
            out0_v[pl.ds(col0, 16)] = a0
            out1_v[pl.ds(col0, 16)] = a1

    pltpu.async_copy(t_hbm, table_v, sem_t)
    pltpu.async_copy(chunk_src(0), idx_a, sem_a)
    pltpu.make_async_copy(t_hbm, table_v, sem_t).wait()

    def pair_body(g, _):
        c0 = g * 2
        pltpu.async_copy(chunk_src(c0 + 1), idx_b, sem_b)
        pltpu.make_async_copy(chunk_src(c0), idx_a, sem_a).wait()
        consume(c0, idx_a)

        @pl.when(g < _NCHUNK // 2 - 1)
        def _():
            pltpu.async_copy(chunk_src(c0 + 2), idx_a, sem_a)

        pltpu.make_async_copy(chunk_src(c0 + 1), idx_b, sem_b).wait()
        consume(c0 + 1, idx_b)
        return ()

    lax.fori_loop(0, _NCHUNK // 2, pair_body, ())

    pltpu.sync_copy(out0_v, out_hbm.at[0, pl.ds(row0, _ROWS_PER_W)])
    pltpu.sync_copy(out1_v, out_hbm.at[1, pl.ds(row0, _ROWS_PER_W)])


@functools.partial(
    pl.kernel,
    mesh=plsc.VectorSubcoreMesh(core_axis_name="c", subcore_axis_name="s"),
    out_type=jax.ShapeDtypeStruct((2, _B), jnp.float32),
    compiler_params=pltpu.CompilerParams(
        needs_layout_passes=False, use_tc_tiling_on_sc=False),
    scratch_types=[
        pltpu.VMEM((_VPAD,), jnp.int32),
        pltpu.VMEM((_LCH, _ROWS_PER_W), jnp.int32),
        pltpu.VMEM((_LCH, _ROWS_PER_W), jnp.int32),
        pltpu.VMEM((_ROWS_PER_W,), jnp.float32),
        pltpu.VMEM((_ROWS_PER_W,), jnp.float32),
        pltpu.SemaphoreType.DMA,
        pltpu.SemaphoreType.DMA,
        pltpu.SemaphoreType.DMA,
    ],
)
def _sc_gather_sum(t_hbm, idx_hbm, out_hbm, table_v, idx_a, idx_b, out0_v,
                   out1_v, sem_a, sem_b, sem_t):
    _sc_body(t_hbm, idx_hbm, out_hbm, table_v, idx_a, idx_b, out0_v, out1_v,
             sem_a, sem_b, sem_t)


def kernel(W, R, feat_idx):
    fused = _fuse_tables(W[:, 0], R[:, 0], R[:, 1])
    out2 = _sc_gather_sum(fused, feat_idx.astype(jnp.int32).T)
    return out2.T
